# baseline (speedup 1.0000x reference)
import functools

import jax
import jax.numpy as jnp
from jax.experimental import pallas as pl
from jax.experimental.pallas import tpu as pltpu

_BLK = 1024


def _moe_block_kernel(x_ref, w_ref, b_ref, g_ref,
                      comb_ref, preds_ref, ps_ref, *, ne, c):
    xb = x_ref[...]
    logits = (
        jnp.dot(xb, w_ref[...], preferred_element_type=jnp.float32)
        + b_ref[...]
    )
    gate = logits[:, ne * c:]
    gmax = jnp.max(gate, axis=-1, keepdims=True)
    iota = jax.lax.broadcasted_iota(jnp.int32, gate.shape, 1)
    min_idx = jnp.min(jnp.where(gate == gmax, iota, ne), axis=-1, keepdims=True)
    onehot = jnp.where(iota == min_idx, 1.0, 0.0)
    for n in range(ne):
        preds_ref[n] = logits[:, n * c:(n + 1) * c]
    lane_expert = jax.lax.broadcasted_iota(jnp.int32, (xb.shape[0], ne * c), 1) // c
    mask = jnp.where(lane_expert == min_idx, 1.0, 0.0)
    sel = jnp.dot(logits[:, :ne * c] * mask, g_ref[...],
                  preferred_element_type=jnp.float32,
                  precision=jax.lax.Precision.HIGHEST)
    comb_ref[...] = jax.nn.softmax(sel, axis=-1)
    ps_ref[...] = jnp.sum(onehot, axis=0, keepdims=True).astype(jnp.int32)[None]


def kernel(x, W_experts, b_experts, W_gate, b_gate):
    B, D = x.shape
    ne, _, c = W_experts.shape
    nblk = B // _BLK
    w_cat = jnp.concatenate(
        [jnp.transpose(W_experts, (1, 0, 2)).reshape(D, ne * c), W_gate], axis=1
    )
    b_cat = jnp.concatenate(
        [b_experts.reshape(1, ne * c), b_gate.reshape(1, ne)], axis=1
    )
    sel_g = (jnp.arange(ne * c)[:, None] % c
             == jnp.arange(c)[None, :]).astype(jnp.float32)

    body = functools.partial(_moe_block_kernel, ne=ne, c=c)
    comb, preds, ps = pl.pallas_call(
        body,
        grid=(nblk,),
        in_specs=[
            pl.BlockSpec((_BLK, D), lambda i: (i, 0)),
            pl.BlockSpec((D, ne * c + ne), lambda i: (0, 0)),
            pl.BlockSpec((1, ne * c + ne), lambda i: (0, 0)),
            pl.BlockSpec((ne * c, c), lambda i: (0, 0)),
        ],
        out_specs=[
            pl.BlockSpec((_BLK, c), lambda i: (i, 0)),
            pl.BlockSpec((ne, _BLK, c), lambda i: (0, i, 0)),
            pl.BlockSpec((1, 1, ne), lambda i: (i, 0, 0)),
        ],
        out_shape=[
            jax.ShapeDtypeStruct((B, c), jnp.float32),
            jax.ShapeDtypeStruct((ne, B, c), jnp.float32),
            jax.ShapeDtypeStruct((nblk, 1, ne), jnp.int32),
        ],
        compiler_params=pltpu.CompilerParams(
            dimension_semantics=("parallel",),
        ),
    )(x, w_cat, b_cat, sel_g)
    return comb, preds, jnp.sum(ps[:, 0, :], axis=0)


# speedup vs baseline: 1.0010x; 1.0010x over previous
import functools

import jax
import jax.numpy as jnp
from jax.experimental import pallas as pl
from jax.experimental.pallas import tpu as pltpu

_BLK = 1024


def _moe_block_kernel(x_ref, w_ref, b_ref, g_ref,
                      comb_ref, preds_ref, ps_ref, *, ne, c):
    xb = x_ref[...]
    logits = (
        jnp.dot(xb, w_ref[...], preferred_element_type=jnp.float32)
        + b_ref[...]
    )
    gate = logits[:, ne * c:]
    gmax = jnp.max(gate, axis=-1, keepdims=True)
    iota = jax.lax.broadcasted_iota(jnp.int32, gate.shape, 1)
    min_idx = jnp.min(jnp.where(gate == gmax, iota, ne), axis=-1, keepdims=True)
    onehot = jnp.where(iota == min_idx, 1.0, 0.0)
    for n in range(ne):
        preds_ref[n] = logits[:, n * c:(n + 1) * c]
    lane_expert = jax.lax.broadcasted_iota(jnp.int32, (xb.shape[0], ne * c), 1) // c
    mask = jnp.where(lane_expert == min_idx, 1.0, 0.0)
    sel = jnp.dot(logits[:, :ne * c] * mask, g_ref[...],
                  preferred_element_type=jnp.float32,
                  precision=jax.lax.Precision.HIGHEST)
    comb_ref[...] = jax.nn.softmax(sel, axis=-1)
    cnt = jnp.sum(onehot, axis=0, keepdims=True).astype(jnp.int32)

    i = pl.program_id(0)

    @pl.when(i == 0)
    def _init():
        ps_ref[...] = cnt

    @pl.when(i > 0)
    def _acc():
        ps_ref[...] = ps_ref[...] + cnt


def kernel(x, W_experts, b_experts, W_gate, b_gate):
    B, D = x.shape
    ne, _, c = W_experts.shape
    nblk = B // _BLK
    w_cat = jnp.concatenate(
        [jnp.transpose(W_experts, (1, 0, 2)).reshape(D, ne * c), W_gate], axis=1
    )
    b_cat = jnp.concatenate(
        [b_experts.reshape(1, ne * c), b_gate.reshape(1, ne)], axis=1
    )
    sel_g = (jnp.arange(ne * c)[:, None] % c
             == jnp.arange(c)[None, :]).astype(jnp.float32)

    body = functools.partial(_moe_block_kernel, ne=ne, c=c)
    comb, preds, ps = pl.pallas_call(
        body,
        grid=(nblk,),
        in_specs=[
            pl.BlockSpec((_BLK, D), lambda i: (i, 0)),
            pl.BlockSpec((D, ne * c + ne), lambda i: (0, 0)),
            pl.BlockSpec((1, ne * c + ne), lambda i: (0, 0)),
            pl.BlockSpec((ne * c, c), lambda i: (0, 0)),
        ],
        out_specs=[
            pl.BlockSpec((_BLK, c), lambda i: (i, 0)),
            pl.BlockSpec((ne, _BLK, c), lambda i: (0, i, 0)),
            pl.BlockSpec((1, ne), lambda i: (0, 0)),
        ],
        out_shape=[
            jax.ShapeDtypeStruct((B, c), jnp.float32),
            jax.ShapeDtypeStruct((ne, B, c), jnp.float32),
            jax.ShapeDtypeStruct((1, ne), jnp.int32),
        ],
        compiler_params=pltpu.CompilerParams(
            dimension_semantics=("arbitrary",),
        ),
    )(x, w_cat, b_cat, sel_g)
    return comb, preds, ps.reshape(ne)


# speedup vs baseline: 1.1663x; 1.1651x over previous
import functools

import jax
import jax.numpy as jnp
from jax.experimental import pallas as pl
from jax.experimental.pallas import tpu as pltpu

_BLK = 1024


def _moe_block_kernel(x_ref, w_ref, b_ref, e_ref, g_ref,
                      comb_ref, preds_ref, ps_ref, *, ne, c):
    xb = x_ref[...]
    logits = (
        jnp.dot(xb, w_ref[...], preferred_element_type=jnp.float32)
        + b_ref[...]
    )
    gate = logits[:, ne * c:]
    gmax = jnp.max(gate, axis=-1, keepdims=True)
    iota = jax.lax.broadcasted_iota(jnp.int32, gate.shape, 1)
    min_idx = jnp.min(jnp.where(gate == gmax, iota, ne), axis=-1, keepdims=True)
    onehot = jnp.where(iota == min_idx, 1.0, 0.0)
    for n in range(ne):
        preds_ref[n] = logits[:, n * c:(n + 1) * c]
    mask = jnp.dot(onehot, e_ref[...], preferred_element_type=jnp.float32,
                   precision=jax.lax.Precision.HIGHEST)
    sel = jnp.dot(logits[:, :ne * c] * mask, g_ref[...],
                  preferred_element_type=jnp.float32,
                  precision=jax.lax.Precision.HIGHEST)
    comb_ref[...] = jax.nn.softmax(sel, axis=-1)
    cnt = jnp.sum(onehot, axis=0, keepdims=True).astype(jnp.int32)

    i = pl.program_id(0)

    @pl.when(i == 0)
    def _init():
        ps_ref[...] = cnt

    @pl.when(i > 0)
    def _acc():
        ps_ref[...] = ps_ref[...] + cnt


def kernel(x, W_experts, b_experts, W_gate, b_gate):
    B, D = x.shape
    ne, _, c = W_experts.shape
    nblk = B // _BLK
    w_cat = jnp.concatenate(
        [jnp.transpose(W_experts, (1, 0, 2)).reshape(D, ne * c), W_gate], axis=1
    )
    b_cat = jnp.concatenate(
        [b_experts.reshape(1, ne * c), b_gate.reshape(1, ne)], axis=1
    )
    sel_e = (jnp.arange(ne * c)[None, :] // c
             == jnp.arange(ne)[:, None]).astype(jnp.float32)
    sel_g = (jnp.arange(ne * c)[:, None] % c
             == jnp.arange(c)[None, :]).astype(jnp.float32)

    body = functools.partial(_moe_block_kernel, ne=ne, c=c)
    comb, preds, ps = pl.pallas_call(
        body,
        grid=(nblk,),
        in_specs=[
            pl.BlockSpec((_BLK, D), lambda i: (i, 0)),
            pl.BlockSpec((D, ne * c + ne), lambda i: (0, 0)),
            pl.BlockSpec((1, ne * c + ne), lambda i: (0, 0)),
            pl.BlockSpec((ne, ne * c), lambda i: (0, 0)),
            pl.BlockSpec((ne * c, c), lambda i: (0, 0)),
        ],
        out_specs=[
            pl.BlockSpec((_BLK, c), lambda i: (i, 0)),
            pl.BlockSpec((ne, _BLK, c), lambda i: (0, i, 0)),
            pl.BlockSpec((1, ne), lambda i: (0, 0)),
        ],
        out_shape=[
            jax.ShapeDtypeStruct((B, c), jnp.float32),
            jax.ShapeDtypeStruct((ne, B, c), jnp.float32),
            jax.ShapeDtypeStruct((1, ne), jnp.int32),
        ],
        compiler_params=pltpu.CompilerParams(
            dimension_semantics=("arbitrary",),
        ),
    )(x, w_cat, b_cat, sel_e, sel_g)
    return comb, preds, ps.reshape(ne)


# speedup vs baseline: 1.2323x; 1.0566x over previous
import functools

import jax
import jax.numpy as jnp
from jax.experimental import pallas as pl
from jax.experimental.pallas import tpu as pltpu

_BLK = 2048


def _moe_block_kernel(x_ref, w_ref, b_ref, e_ref, g_ref,
                      comb_ref, preds_ref, ps_ref, *, ne, c):
    xb = x_ref[...]
    logits = (
        jnp.dot(xb, w_ref[...], preferred_element_type=jnp.float32)
        + b_ref[...]
    )
    gate = logits[:, ne * c:]
    gmax = jnp.max(gate, axis=-1, keepdims=True)
    iota = jax.lax.broadcasted_iota(jnp.int32, gate.shape, 1)
    min_idx = jnp.min(jnp.where(gate == gmax, iota, ne), axis=-1, keepdims=True)
    onehot = jnp.where(iota == min_idx, 1.0, 0.0)
    for n in range(ne):
        preds_ref[n] = logits[:, n * c:(n + 1) * c]
    mask = jnp.dot(onehot, e_ref[...], preferred_element_type=jnp.float32,
                   precision=jax.lax.Precision.HIGHEST)
    sel = jnp.dot(logits[:, :ne * c] * mask, g_ref[...],
                  preferred_element_type=jnp.float32,
                  precision=jax.lax.Precision.HIGHEST)
    comb_ref[...] = jax.nn.softmax(sel, axis=-1)
    cnt = jnp.sum(onehot, axis=0, keepdims=True).astype(jnp.int32)

    i = pl.program_id(0)

    @pl.when(i == 0)
    def _init():
        ps_ref[...] = cnt

    @pl.when(i > 0)
    def _acc():
        ps_ref[...] = ps_ref[...] + cnt


def kernel(x, W_experts, b_experts, W_gate, b_gate):
    B, D = x.shape
    ne, _, c = W_experts.shape
    nblk = B // _BLK
    w_cat = jnp.concatenate(
        [jnp.transpose(W_experts, (1, 0, 2)).reshape(D, ne * c), W_gate], axis=1
    )
    b_cat = jnp.concatenate(
        [b_experts.reshape(1, ne * c), b_gate.reshape(1, ne)], axis=1
    )
    sel_e = (jnp.arange(ne * c)[None, :] // c
             == jnp.arange(ne)[:, None]).astype(jnp.float32)
    sel_g = (jnp.arange(ne * c)[:, None] % c
             == jnp.arange(c)[None, :]).astype(jnp.float32)

    body = functools.partial(_moe_block_kernel, ne=ne, c=c)
    comb, preds, ps = pl.pallas_call(
        body,
        grid=(nblk,),
        in_specs=[
            pl.BlockSpec((_BLK, D), lambda i: (i, 0)),
            pl.BlockSpec((D, ne * c + ne), lambda i: (0, 0)),
            pl.BlockSpec((1, ne * c + ne), lambda i: (0, 0)),
            pl.BlockSpec((ne, ne * c), lambda i: (0, 0)),
            pl.BlockSpec((ne * c, c), lambda i: (0, 0)),
        ],
        out_specs=[
            pl.BlockSpec((_BLK, c), lambda i: (i, 0)),
            pl.BlockSpec((ne, _BLK, c), lambda i: (0, i, 0)),
            pl.BlockSpec((1, ne), lambda i: (0, 0)),
        ],
        out_shape=[
            jax.ShapeDtypeStruct((B, c), jnp.float32),
            jax.ShapeDtypeStruct((ne, B, c), jnp.float32),
            jax.ShapeDtypeStruct((1, ne), jnp.int32),
        ],
        compiler_params=pltpu.CompilerParams(
            dimension_semantics=("arbitrary",),
        ),
    )(x, w_cat, b_cat, sel_e, sel_g)
    return comb, preds, ps.reshape(ne)
